# baseline (speedup 1.0000x reference)
import functools

import jax
import jax.numpy as jnp
from jax import lax
from jax.experimental import pallas as pl
from jax.experimental.pallas import tpu as pltpu
from jax.experimental.pallas import tpu_sc as plsc

N = 10000
E = 320000

NC = 2
NS = 16
NW = NC * NS

BATCH = 128
NB = 80
E_PAD = NB * BATCH * NW

TRASH = N
N_PAD = 10112
STRIPE = N_PAD // NS
STRIPE_LAST = N - 15 * STRIPE

CH1 = (120, 120, 64)
CH2 = (120, 80)
CH3 = (16,)
BM = 400


def _sc_spmm(fc):
  mesh = plsc.VectorSubcoreMesh(core_axis_name="c", subcore_axis_name="s")

  @functools.partial(
      pl.kernel,
      out_type=jax.ShapeDtypeStruct((NC, N, fc), jnp.float32),
      mesh=mesh,
      compiler_params=pltpu.CompilerParams(use_tc_tiling_on_sc=False),
      scratch_types=[
          pltpu.VMEM((NB, 1, BATCH), jnp.int32),
          pltpu.VMEM((NB, 1, BATCH), jnp.int32),
          pltpu.VMEM((BATCH, fc), jnp.float32),
          pltpu.VMEM((BATCH, fc), jnp.float32),
          pltpu.VMEM_SHARED((N_PAD, fc), jnp.float32),
          pltpu.SemaphoreType.DMA,
          pltpu.SemaphoreType.DMA,
          pltpu.SemaphoreType.DMA,
          pltpu.SemaphoreType.DMA,
      ],
  )
  def kern(table, srcs, dsts, zeros, out,
           idx_s, idx_d, rows_a, rows_b, slab, g_a, g_b, s_a, s_b):
    c = lax.axis_index("c")
    s = lax.axis_index("s")
    wid = s * NC + c

    pltpu.sync_copy(srcs.at[wid], idx_s)
    pltpu.sync_copy(dsts.at[wid], idx_d)
    pltpu.sync_copy(zeros, slab.at[pl.ds(s * STRIPE, STRIPE)])
    plsc.subcore_barrier()

    pltpu.async_copy(table.at[idx_s.at[0, 0]], rows_a, g_a)

    def body(t, carry):
      j = 2 * t
      pltpu.make_async_copy(table.at[idx_s.at[j, 0]], rows_a, g_a).wait()
      pltpu.async_copy(rows_a, slab.at[idx_d.at[j, 0]], s_a, add=True)

      @pl.when(t > 0)
      def _():
        pltpu.make_async_copy(rows_b, slab.at[idx_d.at[j - 1, 0]],
                              s_b).wait()

      pltpu.async_copy(table.at[idx_s.at[j + 1, 0]], rows_b, g_b)

      pltpu.make_async_copy(table.at[idx_s.at[j + 1, 0]], rows_b,
                            g_b).wait()
      pltpu.async_copy(rows_b, slab.at[idx_d.at[j + 1, 0]], s_b, add=True)
      pltpu.make_async_copy(rows_a, slab.at[idx_d.at[j, 0]], s_a).wait()

      @pl.when(t < NB // 2 - 1)
      def _():
        pltpu.async_copy(table.at[idx_s.at[j + 2, 0]], rows_a, g_a)

      return carry

    lax.fori_loop(0, NB // 2, body, 0)
    pltpu.make_async_copy(rows_b, slab.at[idx_d.at[NB - 1, 0]],
                          s_b).wait()
    plsc.subcore_barrier()

    @pl.when(s < NS - 1)
    def _full():
      pltpu.sync_copy(slab.at[pl.ds(s * STRIPE, STRIPE)],
                      out.at[c, pl.ds(s * STRIPE, STRIPE)])

    @pl.when(s == NS - 1)
    def _last():
      pltpu.sync_copy(slab.at[pl.ds((NS - 1) * STRIPE, STRIPE_LAST)],
                      out.at[c, pl.ds((NS - 1) * STRIPE, STRIPE_LAST)])

  return kern


def _tc_matmul(x, ws):
  k = x.shape[1]

  def body(x_ref, *refs):
    w_refs, o_refs = refs[:len(ws)], refs[len(ws):]
    xb = x_ref[...]
    for w_ref, o_ref in zip(w_refs, o_refs):
      o_ref[...] = jnp.dot(xb, w_ref[...], preferred_element_type=jnp.float32)

  return pl.pallas_call(
      body,
      grid=(N // BM,),
      in_specs=[pl.BlockSpec((BM, k), lambda i: (i, 0))]
      + [pl.BlockSpec(w.shape, lambda i: (0, 0)) for w in ws],
      out_specs=[pl.BlockSpec((BM, w.shape[1]), lambda i: (i, 0))
                 for w in ws],
      out_shape=[jax.ShapeDtypeStruct((N, w.shape[1]), jnp.float32)
                 for w in ws],
  )(x, *ws)


def _tc_partial(p, b, wlist, acc):
  nw = len(wlist)
  na = 0 if acc is None else len(acc)
  fci = p.shape[2]

  def body(p_ref, b_ref, *refs):
    w_refs = refs[:nw]
    a_refs = refs[nw:nw + na]
    o_refs = refs[nw + na:]
    h = jnp.maximum(p_ref[0] + p_ref[1] + b_ref[0][None, :], 0.0)
    for o in range(nw):
      r = jnp.dot(h, w_refs[o][...], preferred_element_type=jnp.float32)
      if na:
        r = r + a_refs[o][...]
      o_refs[o][...] = r

  args = [p, b] + list(wlist) + (list(acc) if acc else [])
  return pl.pallas_call(
      body,
      grid=(N // BM,),
      in_specs=(
          [pl.BlockSpec((NC, BM, fci), lambda i: (0, i, 0)),
           pl.BlockSpec(b.shape, lambda i: (0, 0))]
          + [pl.BlockSpec(w.shape, lambda i: (0, 0)) for w in wlist]
          + [pl.BlockSpec((BM, a.shape[1]), lambda i: (i, 0))
             for a in (acc or [])]),
      out_specs=[pl.BlockSpec((BM, w.shape[1]), lambda i: (i, 0))
                 for w in wlist],
      out_shape=[jax.ShapeDtypeStruct((N, w.shape[1]), jnp.float32)
                 for w in wlist],
  )(*args)


def _tc_fuse_matmul_split(ps, bs, ws):
  acc = None
  for i in range(len(ps)):
    acc = _tc_partial(ps[i], bs[i], ws[i], acc)
  return acc


def _tc_fuse_matmul(ps, bs, ws):
  ni, no = len(ps), len(ws[0])

  def body(*refs):
    p_refs = refs[:ni]
    b_refs = refs[ni:2 * ni]
    w_refs = refs[2 * ni:2 * ni + ni * no]
    o_refs = refs[2 * ni + ni * no:]
    hs = [jnp.maximum(p[0] + p[1] + b[0][None, :], 0.0)
          for p, b in zip(p_refs, b_refs)]
    for o in range(no):
      acc = jnp.dot(hs[0], w_refs[o][...],
                    preferred_element_type=jnp.float32)
      for i in range(1, ni):
        acc += jnp.dot(hs[i], w_refs[i * no + o][...],
                       preferred_element_type=jnp.float32)
      o_refs[o][...] = acc

  flat_ws = [ws[i][o] for i in range(ni) for o in range(no)]
  return pl.pallas_call(
      body,
      grid=(N // BM,),
      in_specs=(
          [pl.BlockSpec((NC, BM, p.shape[2]), lambda i: (0, i, 0))
           for p in ps]
          + [pl.BlockSpec(b.shape, lambda i: (0, 0)) for b in bs]
          + [pl.BlockSpec(w.shape, lambda i: (0, 0)) for w in flat_ws]),
      out_specs=[pl.BlockSpec((BM, w.shape[1]), lambda i: (i, 0))
                 for w in ws[0]],
      out_shape=[jax.ShapeDtypeStruct((N, w.shape[1]), jnp.float32)
                 for w in ws[0]],
  )(*ps, *bs, *flat_ws)


def _tc_softmax(p, bias, ncls):
  fc = p.shape[2]

  def body(p_ref, b_ref, o_ref):
    v = p_ref[0] + p_ref[1] + b_ref[0][None, :]
    mask = lax.broadcasted_iota(jnp.int32, (BM, fc), 1) < ncls
    v = jnp.where(mask, v, -1e30)
    v = v - jnp.max(v, axis=1, keepdims=True)
    e = jnp.exp(v)
    o_ref[...] = e / jnp.sum(e, axis=1, keepdims=True)

  return pl.pallas_call(
      body,
      grid=(N // BM,),
      in_specs=[
          pl.BlockSpec((NC, BM, fc), lambda i: (0, i, 0)),
          pl.BlockSpec((1, fc), lambda i: (0, 0)),
      ],
      out_specs=pl.BlockSpec((BM, fc), lambda i: (i, 0)),
      out_shape=jax.ShapeDtypeStruct((N, fc), jnp.float32),
  )(p, bias)


def _chunk_cols(w, chunks):
  out, c0 = [], 0
  for fc in chunks:
    blk = w[:, c0:min(c0 + fc, w.shape[1])]
    if blk.shape[1] < fc:
      blk = jnp.pad(blk, ((0, 0), (0, fc - blk.shape[1])))
    out.append(blk)
    c0 += fc
  return out


def _chunk_bias(b, chunks):
  out, c0 = [], 0
  for fc in chunks:
    blk = b[c0:min(c0 + fc, b.shape[0])]
    if blk.shape[0] < fc:
      blk = jnp.pad(blk, (0, fc - blk.shape[0]))
    out.append(blk.reshape(1, fc))
    c0 += fc
  return out


def kernel(x, adj, W1, b1, W2, b2, W3, b3):
  pad = E_PAD - E
  src = jnp.concatenate(
      [adj[0], (jnp.arange(pad, dtype=jnp.int32) * 997) % N])
  dst = jnp.concatenate(
      [adj[1], jnp.full((pad,), TRASH, dtype=jnp.int32)])
  srcs = src.reshape(NW, NB, 1, BATCH)
  dsts = dst.reshape(NW, NB, 1, BATCH)

  zeros = {fc: jnp.zeros((STRIPE, fc), jnp.float32)
           for fc in set(CH1 + CH2 + CH3)}
  spmm = {fc: _sc_spmm(fc) for fc in set(CH1 + CH2 + CH3)}

  w1c = _chunk_cols(W1, CH1)
  b1c = _chunk_bias(b1, CH1)
  w2rows = _chunk_cols(W2.T, CH1)
  w2c = [ _chunk_cols(r.T, CH2) for r in w2rows ]
  b2c = _chunk_bias(b2, CH2)
  w3rows = _chunk_cols(W3.T, CH2)
  w3c = [ _chunk_cols(r.T, CH3) for r in w3rows ]
  b3c = _chunk_bias(b3, CH3)

  s1 = _tc_matmul(x, w1c)
  p1 = [spmm[fc](s, srcs, dsts, zeros[fc]) for fc, s in zip(CH1, s1)]

  s2 = _tc_fuse_matmul_split(p1, b1c, w2c)
  p2 = [spmm[fc](s, srcs, dsts, zeros[fc]) for fc, s in zip(CH2, s2)]

  s3 = _tc_fuse_matmul_split(p2, b2c, w3c)
  p3 = spmm[CH3[0]](s3[0], srcs, dsts, zeros[CH3[0]])

  out = _tc_softmax(p3, b3c[0], ncls=7)
  return out[:, :7]


# speedup vs baseline: 1.0692x; 1.0692x over previous
import functools

import jax
import jax.numpy as jnp
from jax import lax
from jax.experimental import pallas as pl
from jax.experimental.pallas import tpu as pltpu
from jax.experimental.pallas import tpu_sc as plsc

N = 10000
E = 320000

NC = 2
NS = 16
NW = NC * NS

BATCH = 128
NB = 80
E_PAD = NB * BATCH * NW

TRASH = N
N_PAD = 10112
STRIPE = N_PAD // NS
STRIPE_LAST = N - 15 * STRIPE

CH1 = (120, 120, 64)
CH2 = (120, 80)
CH3 = (16,)
BM = 400


def _sc_spmm(fc):
  mesh = plsc.VectorSubcoreMesh(core_axis_name="c", subcore_axis_name="s")

  @functools.partial(
      pl.kernel,
      out_type=jax.ShapeDtypeStruct((NC, N, fc), jnp.float32),
      mesh=mesh,
      compiler_params=pltpu.CompilerParams(use_tc_tiling_on_sc=False),
      scratch_types=[
          pltpu.VMEM((NB, 1, BATCH), jnp.int32),
          pltpu.VMEM((NB, 1, BATCH), jnp.int32),
          pltpu.VMEM((BATCH, fc), jnp.float32),
          pltpu.VMEM((BATCH, fc), jnp.float32),
          pltpu.VMEM_SHARED((N_PAD, fc), jnp.float32),
          pltpu.SemaphoreType.DMA,
          pltpu.SemaphoreType.DMA,
          pltpu.SemaphoreType.DMA,
          pltpu.SemaphoreType.DMA,
      ],
  )
  def kern(table, srcs, dsts, zeros, out,
           idx_s, idx_d, rows_a, rows_b, slab, g_a, g_b, s_a, s_b):
    c = lax.axis_index("c")
    s = lax.axis_index("s")
    wid = s * NC + c

    pltpu.sync_copy(srcs.at[wid], idx_s)
    pltpu.async_copy(table.at[idx_s.at[0, 0]], rows_a, g_a)
    pltpu.sync_copy(dsts.at[wid], idx_d)
    pltpu.sync_copy(zeros, slab.at[pl.ds(s * STRIPE, STRIPE)])
    plsc.subcore_barrier()


    def body(t, carry):
      j = 2 * t
      pltpu.make_async_copy(table.at[idx_s.at[j, 0]], rows_a, g_a).wait()
      pltpu.async_copy(rows_a, slab.at[idx_d.at[j, 0]], s_a, add=True)

      @pl.when(t > 0)
      def _():
        pltpu.make_async_copy(rows_b, slab.at[idx_d.at[j - 1, 0]],
                              s_b).wait()

      pltpu.async_copy(table.at[idx_s.at[j + 1, 0]], rows_b, g_b)

      pltpu.make_async_copy(table.at[idx_s.at[j + 1, 0]], rows_b,
                            g_b).wait()
      pltpu.async_copy(rows_b, slab.at[idx_d.at[j + 1, 0]], s_b, add=True)
      pltpu.make_async_copy(rows_a, slab.at[idx_d.at[j, 0]], s_a).wait()

      @pl.when(t < NB // 2 - 1)
      def _():
        pltpu.async_copy(table.at[idx_s.at[j + 2, 0]], rows_a, g_a)

      return carry

    lax.fori_loop(0, NB // 2, body, 0)
    pltpu.make_async_copy(rows_b, slab.at[idx_d.at[NB - 1, 0]],
                          s_b).wait()
    plsc.subcore_barrier()

    @pl.when(s < NS - 1)
    def _full():
      pltpu.sync_copy(slab.at[pl.ds(s * STRIPE, STRIPE)],
                      out.at[c, pl.ds(s * STRIPE, STRIPE)])

    @pl.when(s == NS - 1)
    def _last():
      pltpu.sync_copy(slab.at[pl.ds((NS - 1) * STRIPE, STRIPE_LAST)],
                      out.at[c, pl.ds((NS - 1) * STRIPE, STRIPE_LAST)])

  return kern


def _tc_matmul(x, ws):
  k = x.shape[1]

  def body(x_ref, *refs):
    w_refs, o_refs = refs[:len(ws)], refs[len(ws):]
    xb = x_ref[...]
    for w_ref, o_ref in zip(w_refs, o_refs):
      o_ref[...] = jnp.dot(xb, w_ref[...], preferred_element_type=jnp.float32)

  return pl.pallas_call(
      body,
      grid=(N // BM,),
      in_specs=[pl.BlockSpec((BM, k), lambda i: (i, 0))]
      + [pl.BlockSpec(w.shape, lambda i: (0, 0)) for w in ws],
      out_specs=[pl.BlockSpec((BM, w.shape[1]), lambda i: (i, 0))
                 for w in ws],
      out_shape=[jax.ShapeDtypeStruct((N, w.shape[1]), jnp.float32)
                 for w in ws],
  )(x, *ws)


def _tc_partial(p, b, wlist, acc):
  nw = len(wlist)
  na = 0 if acc is None else len(acc)
  fci = p.shape[2]

  def body(p_ref, b_ref, *refs):
    w_refs = refs[:nw]
    a_refs = refs[nw:nw + na]
    o_refs = refs[nw + na:]
    h = jnp.maximum(p_ref[0] + p_ref[1] + b_ref[0][None, :], 0.0)
    for o in range(nw):
      r = jnp.dot(h, w_refs[o][...], preferred_element_type=jnp.float32)
      if na:
        r = r + a_refs[o][...]
      o_refs[o][...] = r

  args = [p, b] + list(wlist) + (list(acc) if acc else [])
  return pl.pallas_call(
      body,
      grid=(N // BM,),
      in_specs=(
          [pl.BlockSpec((NC, BM, fci), lambda i: (0, i, 0)),
           pl.BlockSpec(b.shape, lambda i: (0, 0))]
          + [pl.BlockSpec(w.shape, lambda i: (0, 0)) for w in wlist]
          + [pl.BlockSpec((BM, a.shape[1]), lambda i: (i, 0))
             for a in (acc or [])]),
      out_specs=[pl.BlockSpec((BM, w.shape[1]), lambda i: (i, 0))
                 for w in wlist],
      out_shape=[jax.ShapeDtypeStruct((N, w.shape[1]), jnp.float32)
                 for w in wlist],
  )(*args)


def _tc_fuse_matmul_split(ps, bs, ws):
  acc = None
  for i in range(len(ps)):
    acc = _tc_partial(ps[i], bs[i], ws[i], acc)
  return acc


def _tc_fuse_matmul(ps, bs, ws):
  ni, no = len(ps), len(ws[0])

  def body(*refs):
    p_refs = refs[:ni]
    b_refs = refs[ni:2 * ni]
    w_refs = refs[2 * ni:2 * ni + ni * no]
    o_refs = refs[2 * ni + ni * no:]
    hs = [jnp.maximum(p[0] + p[1] + b[0][None, :], 0.0)
          for p, b in zip(p_refs, b_refs)]
    for o in range(no):
      acc = jnp.dot(hs[0], w_refs[o][...],
                    preferred_element_type=jnp.float32)
      for i in range(1, ni):
        acc += jnp.dot(hs[i], w_refs[i * no + o][...],
                       preferred_element_type=jnp.float32)
      o_refs[o][...] = acc

  flat_ws = [ws[i][o] for i in range(ni) for o in range(no)]
  return pl.pallas_call(
      body,
      grid=(N // BM,),
      in_specs=(
          [pl.BlockSpec((NC, BM, p.shape[2]), lambda i: (0, i, 0))
           for p in ps]
          + [pl.BlockSpec(b.shape, lambda i: (0, 0)) for b in bs]
          + [pl.BlockSpec(w.shape, lambda i: (0, 0)) for w in flat_ws]),
      out_specs=[pl.BlockSpec((BM, w.shape[1]), lambda i: (i, 0))
                 for w in ws[0]],
      out_shape=[jax.ShapeDtypeStruct((N, w.shape[1]), jnp.float32)
                 for w in ws[0]],
  )(*ps, *bs, *flat_ws)


def _tc_softmax(p, bias, ncls):
  fc = p.shape[2]

  def body(p_ref, b_ref, o_ref):
    v = p_ref[0] + p_ref[1] + b_ref[0][None, :]
    mask = lax.broadcasted_iota(jnp.int32, (BM, fc), 1) < ncls
    v = jnp.where(mask, v, -1e30)
    v = v - jnp.max(v, axis=1, keepdims=True)
    e = jnp.exp(v)
    o_ref[...] = e / jnp.sum(e, axis=1, keepdims=True)

  return pl.pallas_call(
      body,
      grid=(N // BM,),
      in_specs=[
          pl.BlockSpec((NC, BM, fc), lambda i: (0, i, 0)),
          pl.BlockSpec((1, fc), lambda i: (0, 0)),
      ],
      out_specs=pl.BlockSpec((BM, fc), lambda i: (i, 0)),
      out_shape=jax.ShapeDtypeStruct((N, fc), jnp.float32),
  )(p, bias)


def _chunk_cols(w, chunks):
  out, c0 = [], 0
  for fc in chunks:
    blk = w[:, c0:min(c0 + fc, w.shape[1])]
    if blk.shape[1] < fc:
      blk = jnp.pad(blk, ((0, 0), (0, fc - blk.shape[1])))
    out.append(blk)
    c0 += fc
  return out


def _chunk_bias(b, chunks):
  out, c0 = [], 0
  for fc in chunks:
    blk = b[c0:min(c0 + fc, b.shape[0])]
    if blk.shape[0] < fc:
      blk = jnp.pad(blk, (0, fc - blk.shape[0]))
    out.append(blk.reshape(1, fc))
    c0 += fc
  return out


def kernel(x, adj, W1, b1, W2, b2, W3, b3):
  pad = E_PAD - E
  src = jnp.concatenate(
      [adj[0], (jnp.arange(pad, dtype=jnp.int32) * 997) % N])
  dst = jnp.concatenate(
      [adj[1], jnp.full((pad,), TRASH, dtype=jnp.int32)])
  srcs = src.reshape(NW, NB, 1, BATCH)
  dsts = dst.reshape(NW, NB, 1, BATCH)

  zeros = {fc: jnp.zeros((STRIPE, fc), jnp.float32)
           for fc in set(CH1 + CH2 + CH3)}
  spmm = {fc: _sc_spmm(fc) for fc in set(CH1 + CH2 + CH3)}

  w1c = _chunk_cols(W1, CH1)
  b1c = _chunk_bias(b1, CH1)
  w2rows = _chunk_cols(W2.T, CH1)
  w2c = [ _chunk_cols(r.T, CH2) for r in w2rows ]
  b2c = _chunk_bias(b2, CH2)
  w3rows = _chunk_cols(W3.T, CH2)
  w3c = [ _chunk_cols(r.T, CH3) for r in w3rows ]
  b3c = _chunk_bias(b3, CH3)

  s1 = _tc_matmul(x, w1c)
  p1 = [spmm[fc](s, srcs, dsts, zeros[fc]) for fc, s in zip(CH1, s1)]

  s2 = _tc_fuse_matmul(p1, b1c, w2c)
  p2 = [spmm[fc](s, srcs, dsts, zeros[fc]) for fc, s in zip(CH2, s2)]

  s3 = _tc_fuse_matmul(p2, b2c, w3c)
  p3 = spmm[CH3[0]](s3[0], srcs, dsts, zeros[CH3[0]])

  out = _tc_softmax(p3, b3c[0], ncls=7)
  return out[:, :7]


# speedup vs baseline: 1.0699x; 1.0006x over previous
import functools

import jax
import jax.numpy as jnp
from jax import lax
from jax.experimental import pallas as pl
from jax.experimental.pallas import tpu as pltpu
from jax.experimental.pallas import tpu_sc as plsc

N = 10000
E = 320000

NC = 2
NS = 16
NW = NC * NS

BATCH = 128
NB = 80
E_PAD = NB * BATCH * NW

TRASH = N
N_PAD = 10112
STRIPE = N_PAD // NS
STRIPE_LAST = N - 15 * STRIPE

CH1 = (120, 120, 64)
CH2 = (120, 80)
CH3 = (16,)
BM = 400


def _sc_spmm(fc):
  mesh = plsc.VectorSubcoreMesh(core_axis_name="c", subcore_axis_name="s")

  @functools.partial(
      pl.kernel,
      out_type=jax.ShapeDtypeStruct((NC, N, fc), jnp.float32),
      mesh=mesh,
      compiler_params=pltpu.CompilerParams(use_tc_tiling_on_sc=False),
      scratch_types=[
          pltpu.VMEM((NB, 1, BATCH), jnp.int32),
          pltpu.VMEM((NB, 1, BATCH), jnp.int32),
          pltpu.VMEM((BATCH, fc), jnp.float32),
          pltpu.VMEM((BATCH, fc), jnp.float32),
          pltpu.VMEM_SHARED((N_PAD, fc), jnp.float32),
          pltpu.SemaphoreType.DMA,
          pltpu.SemaphoreType.DMA,
          pltpu.SemaphoreType.DMA,
          pltpu.SemaphoreType.DMA,
      ],
  )
  def kern(table, srcs, dsts, zeros, out,
           idx_s, idx_d, rows_a, rows_b, slab, g_a, g_b, s_a, s_b):
    c = lax.axis_index("c")
    s = lax.axis_index("s")
    wid = s * NC + c

    pltpu.sync_copy(srcs.at[wid], idx_s)
    pltpu.async_copy(table.at[idx_s.at[0, 0]], rows_a, g_a)
    pltpu.sync_copy(dsts.at[wid], idx_d)
    pltpu.sync_copy(zeros, slab.at[pl.ds(s * STRIPE, STRIPE)])
    plsc.subcore_barrier()


    def body(t, carry):
      j = 2 * t
      pltpu.make_async_copy(table.at[idx_s.at[j, 0]], rows_a, g_a).wait()
      pltpu.async_copy(rows_a, slab.at[idx_d.at[j, 0]], s_a, add=True)

      @pl.when(t > 0)
      def _():
        pltpu.make_async_copy(rows_b, slab.at[idx_d.at[j - 1, 0]],
                              s_b).wait()

      pltpu.async_copy(table.at[idx_s.at[j + 1, 0]], rows_b, g_b)

      pltpu.make_async_copy(table.at[idx_s.at[j + 1, 0]], rows_b,
                            g_b).wait()
      pltpu.async_copy(rows_b, slab.at[idx_d.at[j + 1, 0]], s_b, add=True)
      pltpu.make_async_copy(rows_a, slab.at[idx_d.at[j, 0]], s_a).wait()

      @pl.when(t < NB // 2 - 1)
      def _():
        pltpu.async_copy(table.at[idx_s.at[j + 2, 0]], rows_a, g_a)

      return carry

    lax.fori_loop(0, NB // 2, body, 0)
    pltpu.make_async_copy(rows_b, slab.at[idx_d.at[NB - 1, 0]],
                          s_b).wait()
    plsc.subcore_barrier()

    @pl.when(s < NS - 1)
    def _full():
      pltpu.sync_copy(slab.at[pl.ds(s * STRIPE, STRIPE)],
                      out.at[c, pl.ds(s * STRIPE, STRIPE)])

    @pl.when(s == NS - 1)
    def _last():
      pltpu.sync_copy(slab.at[pl.ds((NS - 1) * STRIPE, STRIPE_LAST)],
                      out.at[c, pl.ds((NS - 1) * STRIPE, STRIPE_LAST)])

  return kern


def _tc_matmul(x, ws):
  k = x.shape[1]

  def body(x_ref, *refs):
    w_refs, o_refs = refs[:len(ws)], refs[len(ws):]
    xb = x_ref[...]
    for w_ref, o_ref in zip(w_refs, o_refs):
      o_ref[...] = jnp.dot(xb, w_ref[...], preferred_element_type=jnp.float32)

  return pl.pallas_call(
      body,
      grid=(N // BM,),
      in_specs=[pl.BlockSpec((BM, k), lambda i: (i, 0))]
      + [pl.BlockSpec(w.shape, lambda i: (0, 0)) for w in ws],
      out_specs=[pl.BlockSpec((BM, w.shape[1]), lambda i: (i, 0))
                 for w in ws],
      out_shape=[jax.ShapeDtypeStruct((N, w.shape[1]), jnp.float32)
                 for w in ws],
  )(x, *ws)


def _tc_fuse_matmul(ps, bs, ws):
  ni, no = len(ps), len(ws[0])

  def body(*refs):
    p_refs = refs[:ni]
    b_refs = refs[ni:2 * ni]
    w_refs = refs[2 * ni:2 * ni + ni * no]
    o_refs = refs[2 * ni + ni * no:]
    hs = [jnp.maximum(p[0] + p[1] + b[0][None, :], 0.0)
          for p, b in zip(p_refs, b_refs)]
    for o in range(no):
      acc = jnp.dot(hs[0], w_refs[o][...],
                    preferred_element_type=jnp.float32)
      for i in range(1, ni):
        acc += jnp.dot(hs[i], w_refs[i * no + o][...],
                       preferred_element_type=jnp.float32)
      o_refs[o][...] = acc

  flat_ws = [ws[i][o] for i in range(ni) for o in range(no)]
  return pl.pallas_call(
      body,
      grid=(N // BM,),
      in_specs=(
          [pl.BlockSpec((NC, BM, p.shape[2]), lambda i: (0, i, 0))
           for p in ps]
          + [pl.BlockSpec(b.shape, lambda i: (0, 0)) for b in bs]
          + [pl.BlockSpec(w.shape, lambda i: (0, 0)) for w in flat_ws]),
      out_specs=[pl.BlockSpec((BM, w.shape[1]), lambda i: (i, 0))
                 for w in ws[0]],
      out_shape=[jax.ShapeDtypeStruct((N, w.shape[1]), jnp.float32)
                 for w in ws[0]],
  )(*ps, *bs, *flat_ws)


def _tc_softmax(p, bias, ncls):
  fc = p.shape[2]

  def body(p_ref, b_ref, o_ref):
    v = p_ref[0] + p_ref[1] + b_ref[0][None, :]
    mask = lax.broadcasted_iota(jnp.int32, (BM, fc), 1) < ncls
    v = jnp.where(mask, v, -1e30)
    v = v - jnp.max(v, axis=1, keepdims=True)
    e = jnp.exp(v)
    o_ref[...] = e / jnp.sum(e, axis=1, keepdims=True)

  return pl.pallas_call(
      body,
      grid=(N // BM,),
      in_specs=[
          pl.BlockSpec((NC, BM, fc), lambda i: (0, i, 0)),
          pl.BlockSpec((1, fc), lambda i: (0, 0)),
      ],
      out_specs=pl.BlockSpec((BM, fc), lambda i: (i, 0)),
      out_shape=jax.ShapeDtypeStruct((N, fc), jnp.float32),
  )(p, bias)


def _chunk_cols(w, chunks):
  out, c0 = [], 0
  for fc in chunks:
    blk = w[:, c0:min(c0 + fc, w.shape[1])]
    if blk.shape[1] < fc:
      blk = jnp.pad(blk, ((0, 0), (0, fc - blk.shape[1])))
    out.append(blk)
    c0 += fc
  return out


def _chunk_bias(b, chunks):
  out, c0 = [], 0
  for fc in chunks:
    blk = b[c0:min(c0 + fc, b.shape[0])]
    if blk.shape[0] < fc:
      blk = jnp.pad(blk, (0, fc - blk.shape[0]))
    out.append(blk.reshape(1, fc))
    c0 += fc
  return out


def kernel(x, adj, W1, b1, W2, b2, W3, b3):
  pad = E_PAD - E
  src = jnp.concatenate(
      [adj[0], (jnp.arange(pad, dtype=jnp.int32) * 997) % N])
  dst = jnp.concatenate(
      [adj[1], jnp.full((pad,), TRASH, dtype=jnp.int32)])
  srcs = src.reshape(NW, NB, 1, BATCH)
  dsts = dst.reshape(NW, NB, 1, BATCH)

  zeros = {fc: jnp.zeros((STRIPE, fc), jnp.float32)
           for fc in set(CH1 + CH2 + CH3)}
  spmm = {fc: _sc_spmm(fc) for fc in set(CH1 + CH2 + CH3)}

  w1c = _chunk_cols(W1, CH1)
  b1c = _chunk_bias(b1, CH1)
  w2rows = _chunk_cols(W2.T, CH1)
  w2c = [ _chunk_cols(r.T, CH2) for r in w2rows ]
  b2c = _chunk_bias(b2, CH2)
  w3rows = _chunk_cols(W3.T, CH2)
  w3c = [ _chunk_cols(r.T, CH3) for r in w3rows ]
  b3c = _chunk_bias(b3, CH3)

  s1 = _tc_matmul(x, w1c)
  p1 = [spmm[fc](s, srcs, dsts, zeros[fc]) for fc, s in zip(CH1, s1)]

  s2 = _tc_fuse_matmul(p1, b1c, w2c)
  p2 = [spmm[fc](s, srcs, dsts, zeros[fc]) for fc, s in zip(CH2, s2)]

  s3 = _tc_fuse_matmul(p2, b2c, w3c)
  p3 = spmm[CH3[0]](s3[0], srcs, dsts, zeros[CH3[0]])

  out = _tc_softmax(p3, b3c[0], ncls=7)
  return out[:, :7]
